# baseline (speedup 1.0000x reference)
import functools

import jax
import jax.numpy as jnp
from jax import lax
from jax.experimental import pallas as pl
from jax.experimental.pallas import tpu as pltpu
from jax.experimental.pallas import tpu_sc as plsc

N_NODES = 10000
D_FEAT = 128
N_EDGES = 320000

NS = 16
NC = 2
LANES = 16
E_TILE = N_EDGES // NS
E_HALF = E_TILE // NC
E_COVER = 20096
NP = 10240
STRIPE = NP // NS
U = 10


def _edge_body(edge_hbm, dinv_hbm, tpart_hbm,
               edges_v, cnt_v, cnt2_v, t_v, t2_v, dinv_loc,
               tmp_v, acc_v, dinv_v,
               stage_sh, dinv_sh, sem):
    c = lax.axis_index("c")
    s = lax.axis_index("s")
    ebase = s * E_TILE
    ebase_al = (ebase // 128) * 128
    boff = ebase - ebase_al
    nbase = s * STRIPE

    pltpu.sync_copy(edge_hbm.at[:, pl.ds(ebase_al, E_COVER)], edges_v)

    zeros16 = jnp.zeros((LANES,), jnp.float32)

    def zero_body(i, _):
        for u in range(8):
            off = i * 8 * LANES + u * LANES
            cnt_v[pl.ds(off, LANES)] = zeros16
            cnt2_v[pl.ds(off, LANES)] = zeros16
            t_v[pl.ds(off, LANES)] = zeros16
            t2_v[pl.ds(off, LANES)] = zeros16
        return _
    lax.fori_loop(0, NP // (8 * LANES), zero_body, None)

    ones16 = jnp.ones((LANES,), jnp.float32)

    def cnt_body(i, _):
        for u in range(U):
            off = boff + i * U * LANES + u * LANES
            sv = edges_v[0, pl.ds(off, LANES)]
            dv = edges_v[1, pl.ds(off, LANES)]
            acc = cnt_v if u % 2 == 0 else cnt2_v
            plsc.addupdate_scatter(acc, [dv], ones16, mask=sv != dv)
        return _
    lax.fori_loop(0, E_TILE // (U * LANES), cnt_body, None)

    def fold_cnt(i, _):
        for u in range(8):
            off = i * 8 * LANES + u * LANES
            cnt_v[pl.ds(off, LANES)] = (
                cnt_v[pl.ds(off, LANES)] + cnt2_v[pl.ds(off, LANES)])
        return _
    lax.fori_loop(0, NP // (8 * LANES), fold_cnt, None)
    pltpu.sync_copy(cnt_v, stage_sh.at[s])
    plsc.subcore_barrier()

    pltpu.sync_copy(stage_sh.at[0, pl.ds(nbase, STRIPE)], acc_v)
    for r in range(1, NS):
        pltpu.sync_copy(stage_sh.at[r, pl.ds(nbase, STRIPE)], tmp_v)

        def add_body(i, _, _r=r):
            for u in range(8):
                off = i * 8 * LANES + u * LANES
                acc_v[pl.ds(off, LANES)] = (
                    acc_v[pl.ds(off, LANES)] + tmp_v[pl.ds(off, LANES)])
            return _
        lax.fori_loop(0, STRIPE // (8 * LANES), add_body, None)

    def dinv_body(i, _):
        for u in range(8):
            off = i * 8 * LANES + u * LANES
            xdeg = acc_v[pl.ds(off, LANES)] + jnp.float32(1.0)
            ii = lax.bitcast_convert_type(xdeg, jnp.int32)
            ii = jnp.int32(0x5F3759DF) - (ii >> 1)
            y = lax.bitcast_convert_type(ii, jnp.float32)
            for _unused in range(3):
                y = y * (jnp.float32(1.5) - jnp.float32(0.5) * xdeg * y * y)
            dinv_v[pl.ds(off, LANES)] = y
        return _
    lax.fori_loop(0, STRIPE // (8 * LANES), dinv_body, None)
    pltpu.sync_copy(dinv_v, dinv_sh.at[pl.ds(nbase, STRIPE)])

    @pl.when(c == 0)
    def _write_dinv():
        pltpu.sync_copy(dinv_v, dinv_hbm.at[pl.ds(nbase, STRIPE)])

    plsc.subcore_barrier()
    pltpu.sync_copy(dinv_sh, dinv_loc)

    hoff = boff + c * E_HALF

    UT = 5
    def t_body(i, _):
        for u in range(UT):
            off = hoff + i * UT * LANES + u * LANES
            sv = edges_v[0, pl.ds(off, LANES)]
            dv = edges_v[1, pl.ds(off, LANES)]
            g = plsc.load_gather(dinv_loc, [dv])
            acc = t_v if u % 2 == 0 else t2_v
            plsc.addupdate_scatter(acc, [sv], g, mask=sv != dv)
        return _
    lax.fori_loop(0, E_HALF // (UT * LANES), t_body, None)

    def fold_t(i, _):
        for u in range(8):
            off = i * 8 * LANES + u * LANES
            t_v[pl.ds(off, LANES)] = (
                t_v[pl.ds(off, LANES)] + t2_v[pl.ds(off, LANES)])
        return _
    lax.fori_loop(0, NP // (8 * LANES), fold_t, None)
    pltpu.sync_copy(t_v, stage_sh.at[s])
    plsc.subcore_barrier()

    pltpu.sync_copy(stage_sh.at[0, pl.ds(nbase, STRIPE)], acc_v)
    for r in range(1, NS):
        pltpu.sync_copy(stage_sh.at[r, pl.ds(nbase, STRIPE)], tmp_v)

        def add2_body(i, _, _r=r):
            for u in range(8):
                off = i * 8 * LANES + u * LANES
                acc_v[pl.ds(off, LANES)] = (
                    acc_v[pl.ds(off, LANES)] + tmp_v[pl.ds(off, LANES)])
            return _
        lax.fori_loop(0, STRIPE // (8 * LANES), add2_body, None)

    pltpu.sync_copy(acc_v, tpart_hbm.at[pl.ds(c * NP + nbase, STRIPE)])


_edge_kernel = pl.kernel(
    _edge_body,
    out_type=(
        jax.ShapeDtypeStruct((NP,), jnp.float32),
        jax.ShapeDtypeStruct((NC * NP,), jnp.float32),
    ),
    mesh=plsc.VectorSubcoreMesh(
        core_axis_name="c", subcore_axis_name="s", num_cores=NC),
    compiler_params=pltpu.CompilerParams(needs_layout_passes=False),
    scratch_types=[
        pltpu.VMEM((2, E_COVER), jnp.int32),
        pltpu.VMEM((NP,), jnp.float32),
        pltpu.VMEM((NP,), jnp.float32),
        pltpu.VMEM((NP,), jnp.float32),
        pltpu.VMEM((NP,), jnp.float32),
        pltpu.VMEM((NP,), jnp.float32),
        pltpu.VMEM((STRIPE,), jnp.float32),
        pltpu.VMEM((STRIPE,), jnp.float32),
        pltpu.VMEM((STRIPE,), jnp.float32),
        pltpu.VMEM_SHARED((NS, NP), jnp.float32),
        pltpu.VMEM_SHARED((NP,), jnp.float32),
        pltpu.SemaphoreType.DMA,
    ],
)


def _mm0_body(w0_ref, x_ref, xw_ref):
    xw_ref[...] = lax.dot_general(
        w0_ref[...], x_ref[...], (((1,), (1,)), ((), ())),
        preferred_element_type=jnp.float32)


def _dense_body(xw_ref, b0c_ref, w1_ref, b1_ref, dinv_ref, tp_ref, out_ref):
    dv = dinv_ref[0:1, 0:N_NODES]
    t = tp_ref[0:1, 0:N_NODES] + tp_ref[0:1, NP:NP + N_NODES]
    coef = dv * (t + dv)
    h1t = xw_ref[...] * coef + b0c_ref[...]
    mean = jnp.mean(h1t, axis=1, keepdims=True)
    cen = h1t - mean
    var = jnp.mean(cen * cen, axis=1, keepdims=True)
    hn = cen * lax.rsqrt(var + jnp.float32(1e-5))
    hrt = jnp.maximum(hn, jnp.float32(0.0)) * coef
    out_ref[...] = lax.dot_general(
        hrt, w1_ref[...], (((0,), (1,)), ((), ())),
        preferred_element_type=jnp.float32) + b1_ref[...]


@functools.partial(jax.jit, static_argnames=())
def kernel(x, edge_index, W0, b0, W1, b1):
    dinv_full, t_part = _edge_kernel(edge_index)

    xw0t = pl.pallas_call(
        _mm0_body,
        out_shape=jax.ShapeDtypeStruct((D_FEAT, N_NODES), jnp.float32),
    )(W0, x)

    out = pl.pallas_call(
        _dense_body,
        out_shape=jax.ShapeDtypeStruct((N_NODES, D_FEAT), jnp.float32),
    )(xw0t, b0.reshape(D_FEAT, 1), W1, b1.reshape(1, D_FEAT),
      dinv_full.reshape(1, NP), t_part.reshape(1, NC * NP))
    return out


# speedup vs baseline: 1.1834x; 1.1834x over previous
import functools

import jax
import jax.numpy as jnp
from jax import lax
from jax.experimental import pallas as pl
from jax.experimental.pallas import tpu as pltpu
from jax.experimental.pallas import tpu_sc as plsc

N_NODES = 10000
D_FEAT = 128
N_EDGES = 320000

NS = 16
NC = 2
LANES = 16
E_TILE = N_EDGES // NS
E_HALF = E_TILE // NC
E_COVER = 20096
NP = 10240
STRIPE = NP // NS
U = 10


def _edge_body(edge_hbm, dinv_hbm, tpart_hbm,
               edges_v, cnt_v, t_v, dinv_loc, blk_v, acc_v, dinv_v,
               stage_sh, dinv_sh, sem):
    c = lax.axis_index("c")
    s = lax.axis_index("s")
    ebase = s * E_TILE
    ebase_al = (ebase // 128) * 128
    boff = ebase - ebase_al
    nbase = s * STRIPE

    pltpu.sync_copy(edge_hbm.at[:, pl.ds(ebase_al, E_COVER)], edges_v)

    zeros16 = jnp.zeros((LANES,), jnp.float32)

    def zero_body(i, _):
        for u in range(8):
            off = i * 8 * LANES + u * LANES
            cnt_v[pl.ds(off, LANES)] = zeros16
            t_v[pl.ds(off, LANES)] = zeros16
        return _
    lax.fori_loop(0, NP // (8 * LANES), zero_body, None)

    ones16 = jnp.ones((LANES,), jnp.float32)

    def cnt_body(i, _):
        for u in range(U):
            off = boff + i * U * LANES + u * LANES
            sv = edges_v[0, pl.ds(off, LANES)]
            dv = edges_v[1, pl.ds(off, LANES)]
            plsc.addupdate_scatter(cnt_v, [dv], ones16, mask=sv != dv)
        return _
    lax.fori_loop(0, E_TILE // (U * LANES), cnt_body, None)

    pltpu.sync_copy(cnt_v, stage_sh.at[s])
    plsc.subcore_barrier()
    pltpu.sync_copy(stage_sh.at[:, pl.ds(nbase, STRIPE)], blk_v)

    def red_body(i, _):
        for u in range(4):
            off = i * 4 * LANES + u * LANES
            a = blk_v[0, pl.ds(off, LANES)]
            for r in range(1, NS):
                a = a + blk_v[r, pl.ds(off, LANES)]
            acc_v[pl.ds(off, LANES)] = a
        return _
    lax.fori_loop(0, STRIPE // (4 * LANES), red_body, None)

    def dinv_body(i, _):
        for u in range(8):
            off = i * 8 * LANES + u * LANES
            xdeg = acc_v[pl.ds(off, LANES)] + jnp.float32(1.0)
            ii = lax.bitcast_convert_type(xdeg, jnp.int32)
            ii = jnp.int32(0x5F3759DF) - (ii >> 1)
            y = lax.bitcast_convert_type(ii, jnp.float32)
            for _unused in range(3):
                y = y * (jnp.float32(1.5) - jnp.float32(0.5) * xdeg * y * y)
            dinv_v[pl.ds(off, LANES)] = y
        return _
    lax.fori_loop(0, STRIPE // (8 * LANES), dinv_body, None)
    pltpu.sync_copy(dinv_v, dinv_sh.at[pl.ds(nbase, STRIPE)])

    @pl.when(c == 0)
    def _write_dinv():
        pltpu.sync_copy(dinv_v, dinv_hbm.at[pl.ds(nbase, STRIPE)])

    plsc.subcore_barrier()
    pltpu.sync_copy(dinv_sh, dinv_loc)

    hoff = boff + c * E_HALF

    UT = 5
    def t_body(i, _):
        for u in range(UT):
            off = hoff + i * UT * LANES + u * LANES
            sv = edges_v[0, pl.ds(off, LANES)]
            dv = edges_v[1, pl.ds(off, LANES)]
            g = plsc.load_gather(dinv_loc, [dv])
            plsc.addupdate_scatter(t_v, [sv], g, mask=sv != dv)
        return _
    lax.fori_loop(0, E_HALF // (UT * LANES), t_body, None)

    pltpu.sync_copy(t_v, stage_sh.at[s])
    plsc.subcore_barrier()
    pltpu.sync_copy(stage_sh.at[:, pl.ds(nbase, STRIPE)], blk_v)

    def red2_body(i, _):
        for u in range(4):
            off = i * 4 * LANES + u * LANES
            a = blk_v[0, pl.ds(off, LANES)]
            for r in range(1, NS):
                a = a + blk_v[r, pl.ds(off, LANES)]
            acc_v[pl.ds(off, LANES)] = a
        return _
    lax.fori_loop(0, STRIPE // (4 * LANES), red2_body, None)

    pltpu.sync_copy(acc_v, tpart_hbm.at[pl.ds(c * NP + nbase, STRIPE)])


_edge_kernel = pl.kernel(
    _edge_body,
    out_type=(
        jax.ShapeDtypeStruct((NP,), jnp.float32),
        jax.ShapeDtypeStruct((NC * NP,), jnp.float32),
    ),
    mesh=plsc.VectorSubcoreMesh(
        core_axis_name="c", subcore_axis_name="s", num_cores=NC),
    compiler_params=pltpu.CompilerParams(needs_layout_passes=False),
    scratch_types=[
        pltpu.VMEM((2, E_COVER), jnp.int32),
        pltpu.VMEM((NP,), jnp.float32),
        pltpu.VMEM((NP,), jnp.float32),
        pltpu.VMEM((NP,), jnp.float32),
        pltpu.VMEM((NS, STRIPE), jnp.float32),
        pltpu.VMEM((STRIPE,), jnp.float32),
        pltpu.VMEM((STRIPE,), jnp.float32),
        pltpu.VMEM_SHARED((NS, NP), jnp.float32),
        pltpu.VMEM_SHARED((NP,), jnp.float32),
        pltpu.SemaphoreType.DMA,
    ],
)


def _dense_body(x_ref, w0_ref, b0c_ref, w1_ref, b1_ref, dinv_ref, tp_ref,
                out_ref):
    dv = dinv_ref[0:1, 0:N_NODES]
    t = tp_ref[0:1, 0:N_NODES] + tp_ref[0:1, NP:NP + N_NODES]
    coef = dv * (t + dv)
    h1t = lax.dot_general(
        w0_ref[...], x_ref[...], (((1,), (1,)), ((), ())),
        preferred_element_type=jnp.float32)
    h1t = h1t * coef + b0c_ref[...]
    mean = jnp.mean(h1t, axis=1, keepdims=True)
    cen = h1t - mean
    var = jnp.mean(cen * cen, axis=1, keepdims=True)
    hn = cen * lax.rsqrt(var + jnp.float32(1e-5))
    hrt = jnp.maximum(hn, jnp.float32(0.0)) * coef
    out_ref[...] = lax.dot_general(
        hrt, w1_ref[...], (((0,), (1,)), ((), ())),
        preferred_element_type=jnp.float32) + b1_ref[...]


@functools.partial(jax.jit, static_argnames=())
def kernel(x, edge_index, W0, b0, W1, b1):
    dinv_full, t_part = _edge_kernel(edge_index)

    out = pl.pallas_call(
        _dense_body,
        out_shape=jax.ShapeDtypeStruct((N_NODES, D_FEAT), jnp.float32),
    )(x, W0, b0.reshape(D_FEAT, 1), W1, b1.reshape(1, D_FEAT),
      dinv_full.reshape(1, NP), t_part.reshape(1, NC * NP))
    return out


# speedup vs baseline: 1.2099x; 1.0224x over previous
import functools

import jax
import jax.numpy as jnp
from jax import lax
from jax.experimental import pallas as pl
from jax.experimental.pallas import tpu as pltpu
from jax.experimental.pallas import tpu_sc as plsc

N_NODES = 10000
D_FEAT = 128
N_EDGES = 320000

NS = 16
NC = 2
LANES = 16
E_TILE = N_EDGES // NS
E_HALF = E_TILE // NC
E_COVER = 20096
NP = 10240
STRIPE = NP // NS
U = 10


def _edge_body(edge_hbm, dinv_hbm, tpart_hbm,
               edges_v, cnt_v, t_v, dinv_loc, blk_v, acc_v, dinv_v,
               stage_sh, dinv_sh, sem):
    c = lax.axis_index("c")
    s = lax.axis_index("s")
    ebase = s * E_TILE
    ebase_al = (ebase // 128) * 128
    boff = ebase - ebase_al
    nbase = s * STRIPE

    edge_cp = pltpu.async_copy(
        edge_hbm.at[:, pl.ds(ebase_al, E_COVER)], edges_v, sem)

    zeros16 = jnp.zeros((LANES,), jnp.float32)

    def zero_body(i, _):
        for u in range(8):
            off = i * 8 * LANES + u * LANES
            cnt_v[pl.ds(off, LANES)] = zeros16
            t_v[pl.ds(off, LANES)] = zeros16
        return _
    lax.fori_loop(0, NP // (8 * LANES), zero_body, None)
    edge_cp.wait()

    ones16 = jnp.ones((LANES,), jnp.float32)

    def cnt_body(i, _):
        for u in range(U):
            off = boff + i * U * LANES + u * LANES
            sv = edges_v[0, pl.ds(off, LANES)]
            dv = edges_v[1, pl.ds(off, LANES)]
            plsc.addupdate_scatter(cnt_v, [dv], ones16, mask=sv != dv)
        return _
    lax.fori_loop(0, E_TILE // (U * LANES), cnt_body, None)

    pltpu.sync_copy(cnt_v, stage_sh.at[s])
    plsc.subcore_barrier()
    pltpu.sync_copy(stage_sh.at[:, pl.ds(nbase, STRIPE)], blk_v)

    def red_body(i, _):
        for u in range(4):
            off = i * 4 * LANES + u * LANES
            a = blk_v[0, pl.ds(off, LANES)]
            for r in range(1, NS):
                a = a + blk_v[r, pl.ds(off, LANES)]
            acc_v[pl.ds(off, LANES)] = a
        return _
    lax.fori_loop(0, STRIPE // (4 * LANES), red_body, None)

    def dinv_body(i, _):
        for u in range(8):
            off = i * 8 * LANES + u * LANES
            xdeg = acc_v[pl.ds(off, LANES)] + jnp.float32(1.0)
            ii = lax.bitcast_convert_type(xdeg, jnp.int32)
            ii = jnp.int32(0x5F3759DF) - (ii >> 1)
            y = lax.bitcast_convert_type(ii, jnp.float32)
            for _unused in range(3):
                y = y * (jnp.float32(1.5) - jnp.float32(0.5) * xdeg * y * y)
            dinv_v[pl.ds(off, LANES)] = y
        return _
    lax.fori_loop(0, STRIPE // (8 * LANES), dinv_body, None)
    pltpu.sync_copy(dinv_v, dinv_sh.at[pl.ds(nbase, STRIPE)])

    @pl.when(c == 0)
    def _write_dinv():
        pltpu.sync_copy(dinv_v, dinv_hbm.at[pl.ds(nbase, STRIPE)])

    plsc.subcore_barrier()
    pltpu.sync_copy(dinv_sh, dinv_loc)

    hoff = boff + c * E_HALF

    UT = 5
    def t_body(i, _):
        for u in range(UT):
            off = hoff + i * UT * LANES + u * LANES
            sv = edges_v[0, pl.ds(off, LANES)]
            dv = edges_v[1, pl.ds(off, LANES)]
            g = plsc.load_gather(dinv_loc, [dv])
            plsc.addupdate_scatter(t_v, [sv], g, mask=sv != dv)
        return _
    lax.fori_loop(0, E_HALF // (UT * LANES), t_body, None)

    pltpu.sync_copy(t_v, stage_sh.at[s])
    plsc.subcore_barrier()
    pltpu.sync_copy(stage_sh.at[:, pl.ds(nbase, STRIPE)], blk_v)

    def red2_body(i, _):
        for u in range(4):
            off = i * 4 * LANES + u * LANES
            a = blk_v[0, pl.ds(off, LANES)]
            for r in range(1, NS):
                a = a + blk_v[r, pl.ds(off, LANES)]
            acc_v[pl.ds(off, LANES)] = a
        return _
    lax.fori_loop(0, STRIPE // (4 * LANES), red2_body, None)

    pltpu.sync_copy(acc_v, tpart_hbm.at[pl.ds(c * NP + nbase, STRIPE)])


_edge_kernel = pl.kernel(
    _edge_body,
    out_type=(
        jax.ShapeDtypeStruct((NP,), jnp.float32),
        jax.ShapeDtypeStruct((NC * NP,), jnp.float32),
    ),
    mesh=plsc.VectorSubcoreMesh(
        core_axis_name="c", subcore_axis_name="s", num_cores=NC),
    compiler_params=pltpu.CompilerParams(needs_layout_passes=False),
    scratch_types=[
        pltpu.VMEM((2, E_COVER), jnp.int32),
        pltpu.VMEM((NP,), jnp.float32),
        pltpu.VMEM((NP,), jnp.float32),
        pltpu.VMEM((NP,), jnp.float32),
        pltpu.VMEM((NS, STRIPE), jnp.float32),
        pltpu.VMEM((STRIPE,), jnp.float32),
        pltpu.VMEM((STRIPE,), jnp.float32),
        pltpu.VMEM_SHARED((NS, NP), jnp.float32),
        pltpu.VMEM_SHARED((NP,), jnp.float32),
        pltpu.SemaphoreType.DMA,
    ],
)


def _dense_body(x_ref, w0_ref, b0c_ref, w1_ref, b1_ref, dinv_ref, tp_ref,
                out_ref):
    dv = dinv_ref[0:1, 0:N_NODES]
    t = tp_ref[0:1, 0:N_NODES] + tp_ref[0:1, NP:NP + N_NODES]
    coef = dv * (t + dv)
    h1t = lax.dot_general(
        w0_ref[...], x_ref[...], (((1,), (1,)), ((), ())),
        preferred_element_type=jnp.float32)
    h1t = h1t * coef + b0c_ref[...]
    mean = jnp.mean(h1t, axis=1, keepdims=True)
    m2 = jnp.mean(h1t * h1t, axis=1, keepdims=True)
    var = m2 - mean * mean
    rs = lax.rsqrt(var + jnp.float32(1e-5))
    hn = (h1t - mean) * rs
    hrt = jnp.maximum(hn, jnp.float32(0.0)) * coef
    out_ref[...] = lax.dot_general(
        hrt, w1_ref[...], (((0,), (1,)), ((), ())),
        preferred_element_type=jnp.float32) + b1_ref[...]


@functools.partial(jax.jit, static_argnames=())
def kernel(x, edge_index, W0, b0, W1, b1):
    dinv_full, t_part = _edge_kernel(edge_index)

    out = pl.pallas_call(
        _dense_body,
        out_shape=jax.ShapeDtypeStruct((N_NODES, D_FEAT), jnp.float32),
    )(x, W0, b0.reshape(D_FEAT, 1), W1, b1.reshape(1, D_FEAT),
      dinv_full.reshape(1, NP), t_part.reshape(1, NC * NP))
    return out


# speedup vs baseline: 1.2147x; 1.0039x over previous
import functools

import jax
import jax.numpy as jnp
from jax import lax
from jax.experimental import pallas as pl
from jax.experimental.pallas import tpu as pltpu
from jax.experimental.pallas import tpu_sc as plsc

N_NODES = 10000
D_FEAT = 128
N_EDGES = 320000

NS = 16
NC = 2
LANES = 16
E_TILE = N_EDGES // NS
E_HALF = E_TILE // NC
E_COVER = 20096
NP = 10240
STRIPE = NP // NS
U = 10


def _edge_body(edge_hbm, dinv_hbm, tpart_hbm,
               edges_v, cnt_v, t_v, dinv_loc, blk_v, acc_v, dinv_v,
               stage_sh, dinv_sh, sem):
    c = lax.axis_index("c")
    s = lax.axis_index("s")
    ebase = s * E_TILE
    ebase_al = (ebase // 128) * 128
    boff = ebase - ebase_al
    nbase = s * STRIPE

    edge_cp = pltpu.async_copy(
        edge_hbm.at[:, pl.ds(ebase_al, E_COVER)], edges_v, sem)

    zeros16 = jnp.zeros((LANES,), jnp.float32)

    def zero_body(i, _):
        for u in range(8):
            off = i * 8 * LANES + u * LANES
            cnt_v[pl.ds(off, LANES)] = zeros16
            t_v[pl.ds(off, LANES)] = zeros16
        return _
    lax.fori_loop(0, NP // (8 * LANES), zero_body, None)
    edge_cp.wait()

    ones16 = jnp.ones((LANES,), jnp.float32)

    def cnt_body(i, _):
        for u in range(U):
            off = boff + i * U * LANES + u * LANES
            sv = edges_v[0, pl.ds(off, LANES)]
            dv = edges_v[1, pl.ds(off, LANES)]
            plsc.addupdate_scatter(cnt_v, [dv], ones16, mask=sv != dv)
        return _
    lax.fori_loop(0, E_TILE // (U * LANES), cnt_body, None)

    pltpu.sync_copy(cnt_v, stage_sh.at[s])
    plsc.subcore_barrier()
    pltpu.sync_copy(stage_sh.at[:, pl.ds(nbase, STRIPE)], blk_v)

    def red_body(i, _):
        for u in range(4):
            off = i * 4 * LANES + u * LANES
            a = blk_v[0, pl.ds(off, LANES)]
            for r in range(1, NS):
                a = a + blk_v[r, pl.ds(off, LANES)]
            acc_v[pl.ds(off, LANES)] = a
        return _
    lax.fori_loop(0, STRIPE // (4 * LANES), red_body, None)

    def dinv_body(i, _):
        for u in range(8):
            off = i * 8 * LANES + u * LANES
            xdeg = acc_v[pl.ds(off, LANES)] + jnp.float32(1.0)
            ii = lax.bitcast_convert_type(xdeg, jnp.int32)
            ii = jnp.int32(0x5F3759DF) - (ii >> 1)
            y = lax.bitcast_convert_type(ii, jnp.float32)
            for _unused in range(3):
                y = y * (jnp.float32(1.5) - jnp.float32(0.5) * xdeg * y * y)
            dinv_v[pl.ds(off, LANES)] = y
        return _
    lax.fori_loop(0, STRIPE // (8 * LANES), dinv_body, None)
    pltpu.sync_copy(dinv_v, dinv_sh.at[pl.ds(nbase, STRIPE)])

    @pl.when(c == 0)
    def _write_dinv():
        pltpu.sync_copy(dinv_v, dinv_hbm.at[pl.ds(nbase, STRIPE)])

    plsc.subcore_barrier()
    pltpu.sync_copy(dinv_sh, dinv_loc)

    hoff = boff + c * E_HALF

    UT = 5
    def t_body(i, _):
        for u in range(UT):
            off = hoff + i * UT * LANES + u * LANES
            sv = edges_v[0, pl.ds(off, LANES)]
            dv = edges_v[1, pl.ds(off, LANES)]
            g = plsc.load_gather(dinv_loc, [dv])
            plsc.addupdate_scatter(t_v, [sv], g, mask=sv != dv)
        return _
    lax.fori_loop(0, E_HALF // (UT * LANES), t_body, None)

    pltpu.sync_copy(t_v, stage_sh.at[s])
    plsc.subcore_barrier()
    pltpu.sync_copy(stage_sh.at[:, pl.ds(nbase, STRIPE)], blk_v)

    def red2_body(i, _):
        for u in range(4):
            off = i * 4 * LANES + u * LANES
            a = blk_v[0, pl.ds(off, LANES)]
            for r in range(1, NS):
                a = a + blk_v[r, pl.ds(off, LANES)]
            acc_v[pl.ds(off, LANES)] = a
        return _
    lax.fori_loop(0, STRIPE // (4 * LANES), red2_body, None)

    pltpu.sync_copy(acc_v, tpart_hbm.at[pl.ds(c * NP + nbase, STRIPE)])


_edge_kernel = pl.kernel(
    _edge_body,
    out_type=(
        jax.ShapeDtypeStruct((NP,), jnp.float32),
        jax.ShapeDtypeStruct((NC * NP,), jnp.float32),
    ),
    mesh=plsc.VectorSubcoreMesh(
        core_axis_name="c", subcore_axis_name="s", num_cores=NC),
    compiler_params=pltpu.CompilerParams(needs_layout_passes=False),
    scratch_types=[
        pltpu.VMEM((2, E_COVER), jnp.int32),
        pltpu.VMEM((NP,), jnp.float32),
        pltpu.VMEM((NP,), jnp.float32),
        pltpu.VMEM((NP,), jnp.float32),
        pltpu.VMEM((NS, STRIPE), jnp.float32),
        pltpu.VMEM((STRIPE,), jnp.float32),
        pltpu.VMEM((STRIPE,), jnp.float32),
        pltpu.VMEM_SHARED((NS, NP), jnp.float32),
        pltpu.VMEM_SHARED((NP,), jnp.float32),
        pltpu.SemaphoreType.DMA,
    ],
)


def _dense_body(x_ref, w0_ref, w1_ref, b1_ref, dinv_ref, tp_ref,
                out_ref):
    dv = dinv_ref[0:1, 0:N_NODES]
    t = tp_ref[0:1, 0:N_NODES] + tp_ref[0:1, NP:NP + N_NODES]
    coef = dv * (t + dv)
    h1t = lax.dot_general(
        w0_ref[...], x_ref[...], (((1,), (1,)), ((), ())),
        preferred_element_type=jnp.float32)
    h1t = h1t * coef
    mean = jnp.mean(h1t, axis=1, keepdims=True)
    m2 = jnp.mean(h1t * h1t, axis=1, keepdims=True)
    var = m2 - mean * mean
    rs = lax.rsqrt(var + jnp.float32(1e-5))
    hn = (h1t - mean) * rs
    hrt = jnp.maximum(hn, jnp.float32(0.0)) * coef
    out_ref[...] = lax.dot_general(
        hrt, w1_ref[...], (((0,), (1,)), ((), ())),
        preferred_element_type=jnp.float32) + b1_ref[...]


@functools.partial(jax.jit, static_argnames=())
def kernel(x, edge_index, W0, b0, W1, b1):
    dinv_full, t_part = _edge_kernel(edge_index)

    out = pl.pallas_call(
        _dense_body,
        out_shape=jax.ShapeDtypeStruct((N_NODES, D_FEAT), jnp.float32),
    )(x, W0, W1, b1.reshape(1, D_FEAT),
      dinv_full.reshape(1, NP), t_part.reshape(1, NC * NP))
    return out


# speedup vs baseline: 1.2170x; 1.0019x over previous
import functools

import jax
import jax.numpy as jnp
from jax import lax
from jax.experimental import pallas as pl
from jax.experimental.pallas import tpu as pltpu
from jax.experimental.pallas import tpu_sc as plsc

N_NODES = 10000
D_FEAT = 128
N_EDGES = 320000

NS = 16
NC = 2
LANES = 16
E_TILE = N_EDGES // NS
E_HALF = E_TILE // NC
E_COVER = 20096
NP = 10240
STRIPE = NP // NS
U = 25


def _edge_body(edge_hbm, dinv_hbm, tpart_hbm,
               edges_v, cnt_v, t_v, dinv_loc, blk_v, acc_v, dinv_v,
               stage_sh, dinv_sh, sem):
    c = lax.axis_index("c")
    s = lax.axis_index("s")
    ebase = s * E_TILE
    ebase_al = (ebase // 128) * 128
    boff = ebase - ebase_al
    nbase = s * STRIPE

    edge_cp = pltpu.async_copy(
        edge_hbm.at[:, pl.ds(ebase_al, E_COVER)], edges_v, sem)

    zeros16 = jnp.zeros((LANES,), jnp.float32)

    def zero_body(i, _):
        for u in range(8):
            off = i * 8 * LANES + u * LANES
            cnt_v[pl.ds(off, LANES)] = zeros16
            t_v[pl.ds(off, LANES)] = zeros16
        return _
    lax.fori_loop(0, NP // (8 * LANES), zero_body, None)
    edge_cp.wait()

    ones16 = jnp.ones((LANES,), jnp.float32)

    def cnt_body(i, _):
        for u in range(U):
            off = boff + i * U * LANES + u * LANES
            sv = edges_v[0, pl.ds(off, LANES)]
            dv = edges_v[1, pl.ds(off, LANES)]
            plsc.addupdate_scatter(cnt_v, [dv], ones16, mask=sv != dv)
        return _
    lax.fori_loop(0, E_TILE // (U * LANES), cnt_body, None)

    pltpu.sync_copy(cnt_v, stage_sh.at[s])
    plsc.subcore_barrier()
    pltpu.sync_copy(stage_sh.at[:, pl.ds(nbase, STRIPE)], blk_v)

    def red_body(i, _):
        for u in range(4):
            off = i * 4 * LANES + u * LANES
            a = blk_v[0, pl.ds(off, LANES)]
            for r in range(1, NS):
                a = a + blk_v[r, pl.ds(off, LANES)]
            acc_v[pl.ds(off, LANES)] = a
        return _
    lax.fori_loop(0, STRIPE // (4 * LANES), red_body, None)

    def dinv_body(i, _):
        for u in range(8):
            off = i * 8 * LANES + u * LANES
            xdeg = acc_v[pl.ds(off, LANES)] + jnp.float32(1.0)
            ii = lax.bitcast_convert_type(xdeg, jnp.int32)
            ii = jnp.int32(0x5F3759DF) - (ii >> 1)
            y = lax.bitcast_convert_type(ii, jnp.float32)
            for _unused in range(3):
                y = y * (jnp.float32(1.5) - jnp.float32(0.5) * xdeg * y * y)
            dinv_v[pl.ds(off, LANES)] = y
        return _
    lax.fori_loop(0, STRIPE // (8 * LANES), dinv_body, None)
    pltpu.sync_copy(dinv_v, dinv_sh.at[pl.ds(nbase, STRIPE)])

    @pl.when(c == 0)
    def _write_dinv():
        pltpu.sync_copy(dinv_v, dinv_hbm.at[pl.ds(nbase, STRIPE)])

    plsc.subcore_barrier()
    pltpu.sync_copy(dinv_sh, dinv_loc)

    hoff = boff + c * E_HALF

    UT = 25
    def t_body(i, _):
        for u in range(UT):
            off = hoff + i * UT * LANES + u * LANES
            sv = edges_v[0, pl.ds(off, LANES)]
            dv = edges_v[1, pl.ds(off, LANES)]
            g = plsc.load_gather(dinv_loc, [dv])
            plsc.addupdate_scatter(t_v, [sv], g, mask=sv != dv)
        return _
    lax.fori_loop(0, E_HALF // (UT * LANES), t_body, None)

    pltpu.sync_copy(t_v, stage_sh.at[s])
    plsc.subcore_barrier()
    pltpu.sync_copy(stage_sh.at[:, pl.ds(nbase, STRIPE)], blk_v)

    def red2_body(i, _):
        for u in range(4):
            off = i * 4 * LANES + u * LANES
            a = blk_v[0, pl.ds(off, LANES)]
            for r in range(1, NS):
                a = a + blk_v[r, pl.ds(off, LANES)]
            acc_v[pl.ds(off, LANES)] = a
        return _
    lax.fori_loop(0, STRIPE // (4 * LANES), red2_body, None)

    pltpu.sync_copy(acc_v, tpart_hbm.at[pl.ds(c * NP + nbase, STRIPE)])


_edge_kernel = pl.kernel(
    _edge_body,
    out_type=(
        jax.ShapeDtypeStruct((NP,), jnp.float32),
        jax.ShapeDtypeStruct((NC * NP,), jnp.float32),
    ),
    mesh=plsc.VectorSubcoreMesh(
        core_axis_name="c", subcore_axis_name="s", num_cores=NC),
    compiler_params=pltpu.CompilerParams(needs_layout_passes=False),
    scratch_types=[
        pltpu.VMEM((2, E_COVER), jnp.int32),
        pltpu.VMEM((NP,), jnp.float32),
        pltpu.VMEM((NP,), jnp.float32),
        pltpu.VMEM((NP,), jnp.float32),
        pltpu.VMEM((NS, STRIPE), jnp.float32),
        pltpu.VMEM((STRIPE,), jnp.float32),
        pltpu.VMEM((STRIPE,), jnp.float32),
        pltpu.VMEM_SHARED((NS, NP), jnp.float32),
        pltpu.VMEM_SHARED((NP,), jnp.float32),
        pltpu.SemaphoreType.DMA,
    ],
)


def _dense_body(x_ref, w0_ref, w1_ref, b1_ref, dinv_ref, tp_ref,
                out_ref):
    dv = dinv_ref[0:1, 0:N_NODES]
    t = tp_ref[0:1, 0:N_NODES] + tp_ref[0:1, NP:NP + N_NODES]
    coef = dv * (t + dv)
    h1t = lax.dot_general(
        w0_ref[...], x_ref[...], (((1,), (1,)), ((), ())),
        preferred_element_type=jnp.float32)
    h1t = h1t * coef
    mean = jnp.mean(h1t, axis=1, keepdims=True)
    m2 = jnp.mean(h1t * h1t, axis=1, keepdims=True)
    var = m2 - mean * mean
    rs = lax.rsqrt(var + jnp.float32(1e-5))
    hn = (h1t - mean) * rs
    hrt = jnp.maximum(hn, jnp.float32(0.0)) * coef
    out_ref[...] = lax.dot_general(
        hrt, w1_ref[...], (((0,), (1,)), ((), ())),
        preferred_element_type=jnp.float32) + b1_ref[...]


@functools.partial(jax.jit, static_argnames=())
def kernel(x, edge_index, W0, b0, W1, b1):
    dinv_full, t_part = _edge_kernel(edge_index)

    out = pl.pallas_call(
        _dense_body,
        out_shape=jax.ShapeDtypeStruct((N_NODES, D_FEAT), jnp.float32),
    )(x, W0, W1, b1.reshape(1, D_FEAT),
      dinv_full.reshape(1, NP), t_part.reshape(1, NC * NP))
    return out
